# baseline (speedup 1.0000x reference)
import functools

import jax
import jax.numpy as jnp
from jax import lax
from jax.experimental import pallas as pl
from jax.experimental.pallas import tpu as pltpu
from jax.experimental.pallas import tpu_sc as plsc

N_NODES = 10000
N_PAD = 10240
DIM = 128
NUM_EDGES = 320000

NC, NS = 2, 16
NW = NC * NS
EDGES_PER_W = NUM_EDGES // NW
CHUNK = 128
NCHUNK = EDGES_PER_W // CHUNK
TAIL = EDGES_PER_W - NCHUNK * CHUNK
ROWS_PER_SUB = N_PAD // NS

@functools.cache
def _make_segment_sum_sc():
    mesh = plsc.VectorSubcoreMesh(core_axis_name="c", subcore_axis_name="s",
                                  num_cores=NC, num_subcores=NS)

    @functools.partial(
        pl.kernel,
        out_type=jax.ShapeDtypeStruct((NC, N_PAD, DIM), jnp.float32),
        mesh=mesh,
        scratch_types=[
            pltpu.VMEM((3, CHUNK), jnp.int32),
            pltpu.VMEM((3, CHUNK), jnp.int32),
            pltpu.VMEM((2, CHUNK, DIM), jnp.float32),
            pltpu.VMEM((TAIL,), jnp.int32),
            pltpu.VMEM((TAIL,), jnp.int32),
            pltpu.VMEM((TAIL, DIM), jnp.float32),
            pltpu.VMEM_SHARED((N_PAD, DIM), jnp.float32),
            pltpu.SemaphoreType.DMA,
            pltpu.SemaphoreType.DMA,
            pltpu.SemaphoreType.DMA,
            pltpu.SemaphoreType.DMA,
            pltpu.SemaphoreType.DMA,
            pltpu.SemaphoreType.DMA,
            pltpu.SemaphoreType.DMA,
        ],
    )
    def seg_sum(src_hbm, dst_hbm, table_hbm, zeros_hbm, out_hbm,
                src_v, dst_v, rows_v, src_t, dst_t, rows_t, acc_sh,
                semi0, semi1, semi2, semg0, semg1, sems0, sems1):
        cid = lax.axis_index("c")
        sid = lax.axis_index("s")
        wid = sid * NC + cid
        row0 = sid * ROWS_PER_SUB
        sem_i = (semi0, semi1, semi2)
        sem_g = (semg0, semg1)
        sem_s = (sems0, sems1)
        ebase = wid * EDGES_PER_W

        def idx_fetch(ci, s, sem):
            pltpu.async_copy(src_hbm.at[pl.ds(ebase + ci * CHUNK, CHUNK)],
                             src_v.at[s], sem)
            pltpu.async_copy(dst_hbm.at[pl.ds(ebase + ci * CHUNK, CHUNK)],
                             dst_v.at[s], sem)

        def idx_wait(s, sem):
            pltpu.make_async_copy(src_hbm.at[pl.ds(0, CHUNK)],
                                  src_v.at[s], sem).wait()
            pltpu.make_async_copy(dst_hbm.at[pl.ds(0, CHUNK)],
                                  dst_v.at[s], sem).wait()

        half = CHUNK // 2

        def gather(b, s, sem):
            pltpu.async_copy(table_hbm.at[src_v.at[s, pl.ds(0, half)]],
                             rows_v.at[b, pl.ds(0, half)], sem)
            pltpu.async_copy(table_hbm.at[src_v.at[s, pl.ds(half, half)]],
                             rows_v.at[b, pl.ds(half, half)], sem)

        def gather_wait(b, s, sem):
            pltpu.make_async_copy(table_hbm.at[src_v.at[s, pl.ds(0, half)]],
                                  rows_v.at[b, pl.ds(0, half)], sem).wait()
            pltpu.make_async_copy(table_hbm.at[src_v.at[s, pl.ds(half, half)]],
                                  rows_v.at[b, pl.ds(half, half)], sem).wait()

        def scatter(b, s, sem):
            pltpu.async_copy(rows_v.at[b], acc_sh.at[dst_v.at[s]], sem,
                             add=True)

        def scatter_wait(b, s, sem):
            pltpu.make_async_copy(rows_v.at[b], acc_sh.at[dst_v.at[s]],
                                  sem).wait()

        pltpu.sync_copy(zeros_hbm.at[pl.ds(row0, ROWS_PER_SUB)],
                        acc_sh.at[pl.ds(row0, ROWS_PER_SUB)])
        plsc.subcore_barrier()

        idx_fetch(0, 0, sem_i[0])
        idx_wait(0, sem_i[0])
        gather(0, 0, sem_g[0])
        idx_fetch(1, 1, sem_i[1])

        def body(i, carry):
            for k in range(6):
                ci = i * 6 + k
                b, nb = k % 2, (k + 1) % 2
                s, ns, ps = k % 3, (k + 1) % 3, (k + 2) % 3

                @pl.when(ci + 1 < NCHUNK)
                def _():
                    idx_wait(ns, sem_i[ns])

                @pl.when(ci >= 1)
                def _():
                    scatter_wait(nb, ps, sem_s[nb])

                @pl.when(ci + 1 < NCHUNK)
                def _():
                    gather(nb, ns, sem_g[nb])

                gather_wait(b, s, sem_g[b])
                scatter(b, s, sem_s[b])

                @pl.when(ci + 2 < NCHUNK)
                def _():
                    idx_fetch(ci + 2, ps, sem_i[ps])
            return carry

        lax.fori_loop(0, NCHUNK // 6, body, 0)
        scatter_wait(1, 2, sem_s[1])
        tbase = ebase + NCHUNK * CHUNK
        pltpu.sync_copy(src_hbm.at[pl.ds(tbase, TAIL)], src_t)
        pltpu.sync_copy(dst_hbm.at[pl.ds(tbase, TAIL)], dst_t)
        pltpu.async_copy(table_hbm.at[src_t], rows_t, sem_g[0]).wait()
        pltpu.sync_copy(rows_t, acc_sh.at[dst_t], add=True)
        plsc.subcore_barrier()

        pltpu.sync_copy(acc_sh.at[pl.ds(row0, ROWS_PER_SUB)],
                        out_hbm.at[cid, pl.ds(row0, ROWS_PER_SUB)])

    return seg_sum


BR = 1000
GRID = N_NODES // BR


def _mm1_body(acc_ref, w_ref, b_ref, h_ref, stats_ref):
    i = pl.program_id(0)
    a = acc_ref[0] + acc_ref[1]
    h = jnp.dot(a, w_ref[...], preferred_element_type=jnp.float32) + b_ref[...]
    h_ref[...] = h

    @pl.when(i == 0)
    def _():
        stats_ref[...] = jnp.zeros_like(stats_ref)

    stats_ref[0:1, :] += jnp.sum(h, axis=0, keepdims=True)
    stats_ref[1:2, :] += jnp.sum(h * h, axis=0, keepdims=True)


_mm1 = pl.pallas_call(
    _mm1_body,
    grid=(GRID,),
    in_specs=[
        pl.BlockSpec((NC, BR, DIM), lambda i: (0, i, 0)),
        pl.BlockSpec((DIM, DIM), lambda i: (0, 0)),
        pl.BlockSpec((1, DIM), lambda i: (0, 0)),
    ],
    out_specs=[
        pl.BlockSpec((BR, DIM), lambda i: (i, 0)),
        pl.BlockSpec((2, DIM), lambda i: (0, 0)),
    ],
    out_shape=[
        jax.ShapeDtypeStruct((N_NODES, DIM), jnp.float32),
        jax.ShapeDtypeStruct((2, DIM), jnp.float32),
    ],
)


def _bn_relu_body(h_ref, stats_ref, gamma_ref, beta_ref, o_ref):
    mean = stats_ref[0:1, :] / N_NODES
    var = stats_ref[1:2, :] / N_NODES - mean * mean
    rstd = lax.rsqrt(var + 1e-5)
    o_ref[...] = jnp.maximum(
        (h_ref[...] - mean) * (rstd * gamma_ref[...]) + beta_ref[...], 0.0)


_bn_relu = pl.pallas_call(
    _bn_relu_body,
    grid=(GRID,),
    in_specs=[
        pl.BlockSpec((BR, DIM), lambda i: (i, 0)),
        pl.BlockSpec((2, DIM), lambda i: (0, 0)),
        pl.BlockSpec((1, DIM), lambda i: (0, 0)),
        pl.BlockSpec((1, DIM), lambda i: (0, 0)),
    ],
    out_specs=pl.BlockSpec((BR, DIM), lambda i: (i, 0)),
    out_shape=jax.ShapeDtypeStruct((N_NODES, DIM), jnp.float32),
)


def _final_body(acc_ref, h1_ref, w2_ref, b2_ref, wfc_ref, bfc_ref, o_ref):
    a = acc_ref[0] + acc_ref[1]
    h2 = jnp.dot(a, w2_ref[...], preferred_element_type=jnp.float32) + b2_ref[...]
    hjk = jnp.maximum(h1_ref[...], h2)
    o_ref[...] = jnp.dot(hjk, wfc_ref[...],
                         preferred_element_type=jnp.float32) + bfc_ref[...]


_final = pl.pallas_call(
    _final_body,
    grid=(GRID,),
    in_specs=[
        pl.BlockSpec((NC, BR, DIM), lambda i: (0, i, 0)),
        pl.BlockSpec((BR, DIM), lambda i: (i, 0)),
        pl.BlockSpec((DIM, DIM), lambda i: (0, 0)),
        pl.BlockSpec((1, DIM), lambda i: (0, 0)),
        pl.BlockSpec((DIM, DIM), lambda i: (0, 0)),
        pl.BlockSpec((1, DIM), lambda i: (0, 0)),
    ],
    out_specs=pl.BlockSpec((BR, DIM), lambda i: (i, 0)),
    out_shape=jax.ShapeDtypeStruct((N_NODES, DIM), jnp.float32),
)


def kernel(x, edge_index, W1, b1, gamma, beta, W2, b2, Wfc, bfc):
    src = edge_index[0]
    dst = edge_index[1]
    zeros = jnp.zeros((N_PAD, DIM), jnp.float32)

    seg_sum = _make_segment_sum_sc()
    acc1 = seg_sum(src, dst, x, zeros)
    h, stats = _mm1(acc1, W1, b1.reshape(1, DIM))
    h1 = _bn_relu(h, stats, gamma.reshape(1, DIM), beta.reshape(1, DIM))
    acc2 = seg_sum(src, dst, h1, zeros)
    out = _final(acc2, h1, W2, b2.reshape(1, DIM), Wfc, bfc.reshape(1, DIM))
    return out


# speedup vs baseline: 1.0103x; 1.0103x over previous
import functools

import jax
import jax.numpy as jnp
from jax import lax
from jax.experimental import pallas as pl
from jax.experimental.pallas import tpu as pltpu
from jax.experimental.pallas import tpu_sc as plsc

N_NODES = 10000
N_PAD = 10240
DIM = 128
NUM_EDGES = 320000

NC, NS = 2, 16
NW = NC * NS
EDGES_PER_W = NUM_EDGES // NW
CHUNK = 128
NCHUNK = EDGES_PER_W // CHUNK
TAIL = EDGES_PER_W - NCHUNK * CHUNK
ROWS_PER_SUB = N_PAD // NS

@functools.cache
def _make_segment_sum_sc():
    mesh = plsc.VectorSubcoreMesh(core_axis_name="c", subcore_axis_name="s",
                                  num_cores=NC, num_subcores=NS)

    @functools.partial(
        pl.kernel,
        out_type=jax.ShapeDtypeStruct((NC, N_PAD, DIM), jnp.float32),
        mesh=mesh,
        scratch_types=[
            pltpu.VMEM((3, CHUNK), jnp.int32),
            pltpu.VMEM((3, CHUNK), jnp.int32),
            pltpu.VMEM((2, CHUNK, DIM), jnp.float32),
            pltpu.VMEM((TAIL,), jnp.int32),
            pltpu.VMEM((TAIL,), jnp.int32),
            pltpu.VMEM((TAIL, DIM), jnp.float32),
            pltpu.VMEM_SHARED((N_PAD, DIM), jnp.float32),
            pltpu.SemaphoreType.DMA,
            pltpu.SemaphoreType.DMA,
            pltpu.SemaphoreType.DMA,
            pltpu.SemaphoreType.DMA,
            pltpu.SemaphoreType.DMA,
            pltpu.SemaphoreType.DMA,
            pltpu.SemaphoreType.DMA,
            pltpu.SemaphoreType.DMA,
        ],
    )
    def seg_sum(src_hbm, dst_hbm, table_hbm, zeros_hbm, out_hbm,
                src_v, dst_v, rows_v, src_t, dst_t, rows_t, acc_sh,
                semi0, semi1, semi2, semg0, semg1, sems0, sems1, semz):
        cid = lax.axis_index("c")
        sid = lax.axis_index("s")
        wid = sid * NC + cid
        row0 = sid * ROWS_PER_SUB
        sem_i = (semi0, semi1, semi2)
        sem_g = (semg0, semg1)
        sem_s = (sems0, sems1)
        ebase = wid * EDGES_PER_W

        def idx_fetch(ci, s, sem):
            pltpu.async_copy(src_hbm.at[pl.ds(ebase + ci * CHUNK, CHUNK)],
                             src_v.at[s], sem)
            pltpu.async_copy(dst_hbm.at[pl.ds(ebase + ci * CHUNK, CHUNK)],
                             dst_v.at[s], sem)

        def idx_wait(s, sem):
            pltpu.make_async_copy(src_hbm.at[pl.ds(0, CHUNK)],
                                  src_v.at[s], sem).wait()
            pltpu.make_async_copy(dst_hbm.at[pl.ds(0, CHUNK)],
                                  dst_v.at[s], sem).wait()

        def gather(b, s, sem):
            pltpu.async_copy(table_hbm.at[src_v.at[s]], rows_v.at[b], sem)

        def gather_wait(b, s, sem):
            pltpu.make_async_copy(table_hbm.at[src_v.at[s]],
                                  rows_v.at[b], sem).wait()

        def scatter(b, s, sem):
            pltpu.async_copy(rows_v.at[b], acc_sh.at[dst_v.at[s]], sem,
                             add=True)

        def scatter_wait(b, s, sem):
            pltpu.make_async_copy(rows_v.at[b], acc_sh.at[dst_v.at[s]],
                                  sem).wait()

        zcp = pltpu.async_copy(zeros_hbm.at[pl.ds(row0, ROWS_PER_SUB)],
                               acc_sh.at[pl.ds(row0, ROWS_PER_SUB)], semz)

        idx_fetch(0, 0, sem_i[0])
        idx_wait(0, sem_i[0])
        gather(0, 0, sem_g[0])
        idx_fetch(1, 1, sem_i[1])
        zcp.wait()
        plsc.subcore_barrier()

        def body(i, carry):
            for k in range(6):
                ci = i * 6 + k
                b, nb = k % 2, (k + 1) % 2
                s, ns, ps = k % 3, (k + 1) % 3, (k + 2) % 3

                @pl.when(ci + 1 < NCHUNK)
                def _():
                    idx_wait(ns, sem_i[ns])

                @pl.when(ci >= 1)
                def _():
                    scatter_wait(nb, ps, sem_s[nb])

                @pl.when(ci + 1 < NCHUNK)
                def _():
                    gather(nb, ns, sem_g[nb])

                gather_wait(b, s, sem_g[b])
                scatter(b, s, sem_s[b])

                @pl.when(ci + 2 < NCHUNK)
                def _():
                    idx_fetch(ci + 2, ps, sem_i[ps])
            return carry

        lax.fori_loop(0, NCHUNK // 6, body, 0)
        scatter_wait(1, 2, sem_s[1])
        tbase = ebase + NCHUNK * CHUNK
        pltpu.sync_copy(src_hbm.at[pl.ds(tbase, TAIL)], src_t)
        pltpu.sync_copy(dst_hbm.at[pl.ds(tbase, TAIL)], dst_t)
        pltpu.async_copy(table_hbm.at[src_t], rows_t, sem_g[0]).wait()
        pltpu.sync_copy(rows_t, acc_sh.at[dst_t], add=True)
        plsc.subcore_barrier()

        pltpu.sync_copy(acc_sh.at[pl.ds(row0, ROWS_PER_SUB)],
                        out_hbm.at[cid, pl.ds(row0, ROWS_PER_SUB)])

    return seg_sum


BR = 1000
GRID = N_NODES // BR


def _mm1_body(acc_ref, w_ref, b_ref, h_ref, stats_ref):
    i = pl.program_id(0)
    a = acc_ref[0] + acc_ref[1]
    h = jnp.dot(a, w_ref[...], preferred_element_type=jnp.float32) + b_ref[...]
    h_ref[...] = h

    @pl.when(i == 0)
    def _():
        stats_ref[...] = jnp.zeros_like(stats_ref)

    stats_ref[0:1, :] += jnp.sum(h, axis=0, keepdims=True)
    stats_ref[1:2, :] += jnp.sum(h * h, axis=0, keepdims=True)


_mm1 = pl.pallas_call(
    _mm1_body,
    grid=(GRID,),
    in_specs=[
        pl.BlockSpec((NC, BR, DIM), lambda i: (0, i, 0)),
        pl.BlockSpec((DIM, DIM), lambda i: (0, 0)),
        pl.BlockSpec((1, DIM), lambda i: (0, 0)),
    ],
    out_specs=[
        pl.BlockSpec((BR, DIM), lambda i: (i, 0)),
        pl.BlockSpec((2, DIM), lambda i: (0, 0)),
    ],
    out_shape=[
        jax.ShapeDtypeStruct((N_NODES, DIM), jnp.float32),
        jax.ShapeDtypeStruct((2, DIM), jnp.float32),
    ],
)


def _bn_relu_body(h_ref, stats_ref, gamma_ref, beta_ref, o_ref):
    mean = stats_ref[0:1, :] / N_NODES
    var = stats_ref[1:2, :] / N_NODES - mean * mean
    rstd = lax.rsqrt(var + 1e-5)
    o_ref[...] = jnp.maximum(
        (h_ref[...] - mean) * (rstd * gamma_ref[...]) + beta_ref[...], 0.0)


_bn_relu = pl.pallas_call(
    _bn_relu_body,
    grid=(GRID,),
    in_specs=[
        pl.BlockSpec((BR, DIM), lambda i: (i, 0)),
        pl.BlockSpec((2, DIM), lambda i: (0, 0)),
        pl.BlockSpec((1, DIM), lambda i: (0, 0)),
        pl.BlockSpec((1, DIM), lambda i: (0, 0)),
    ],
    out_specs=pl.BlockSpec((BR, DIM), lambda i: (i, 0)),
    out_shape=jax.ShapeDtypeStruct((N_NODES, DIM), jnp.float32),
)


def _final_body(acc_ref, h1_ref, w2_ref, b2_ref, wfc_ref, bfc_ref, o_ref):
    a = acc_ref[0] + acc_ref[1]
    h2 = jnp.dot(a, w2_ref[...], preferred_element_type=jnp.float32) + b2_ref[...]
    hjk = jnp.maximum(h1_ref[...], h2)
    o_ref[...] = jnp.dot(hjk, wfc_ref[...],
                         preferred_element_type=jnp.float32) + bfc_ref[...]


_final = pl.pallas_call(
    _final_body,
    grid=(GRID,),
    in_specs=[
        pl.BlockSpec((NC, BR, DIM), lambda i: (0, i, 0)),
        pl.BlockSpec((BR, DIM), lambda i: (i, 0)),
        pl.BlockSpec((DIM, DIM), lambda i: (0, 0)),
        pl.BlockSpec((1, DIM), lambda i: (0, 0)),
        pl.BlockSpec((DIM, DIM), lambda i: (0, 0)),
        pl.BlockSpec((1, DIM), lambda i: (0, 0)),
    ],
    out_specs=pl.BlockSpec((BR, DIM), lambda i: (i, 0)),
    out_shape=jax.ShapeDtypeStruct((N_NODES, DIM), jnp.float32),
)


def kernel(x, edge_index, W1, b1, gamma, beta, W2, b2, Wfc, bfc):
    src = edge_index[0]
    dst = edge_index[1]
    zeros = jnp.zeros((N_PAD, DIM), jnp.float32)

    seg_sum = _make_segment_sum_sc()
    acc1 = seg_sum(src, dst, x, zeros)
    h, stats = _mm1(acc1, W1, b1.reshape(1, DIM))
    h1 = _bn_relu(h, stats, gamma.reshape(1, DIM), beta.reshape(1, DIM))
    acc2 = seg_sum(src, dst, h1, zeros)
    out = _final(acc2, h1, W2, b2.reshape(1, DIM), Wfc, bfc.reshape(1, DIM))
    return out


# speedup vs baseline: 1.0279x; 1.0174x over previous
import functools

import jax
import jax.numpy as jnp
from jax import lax
from jax.experimental import pallas as pl
from jax.experimental.pallas import tpu as pltpu
from jax.experimental.pallas import tpu_sc as plsc

N_NODES = 10000
N_PAD = 10240
DIM = 128
NUM_EDGES = 320000

NC, NS = 2, 16
NW = NC * NS
EDGES_PER_W = NUM_EDGES // NW
CHUNK = 128
NCHUNK = EDGES_PER_W // CHUNK
TAIL = EDGES_PER_W - NCHUNK * CHUNK
ROWS_PER_SUB = N_PAD // NS

@functools.cache
def _make_segment_sum_sc():
    mesh = plsc.VectorSubcoreMesh(core_axis_name="c", subcore_axis_name="s",
                                  num_cores=NC, num_subcores=NS)

    @functools.partial(
        pl.kernel,
        out_type=jax.ShapeDtypeStruct((NC, N_PAD, DIM), jnp.float32),
        mesh=mesh,
        scratch_types=[
            pltpu.VMEM((3, CHUNK), jnp.int32),
            pltpu.VMEM((3, CHUNK), jnp.int32),
            pltpu.VMEM((2, CHUNK, DIM), jnp.float32),
            pltpu.VMEM((TAIL,), jnp.int32),
            pltpu.VMEM((TAIL,), jnp.int32),
            pltpu.VMEM((TAIL, DIM), jnp.float32),
            pltpu.VMEM_SHARED((N_PAD, DIM), jnp.float32),
            pltpu.SemaphoreType.DMA,
            pltpu.SemaphoreType.DMA,
            pltpu.SemaphoreType.DMA,
            pltpu.SemaphoreType.DMA,
            pltpu.SemaphoreType.DMA,
            pltpu.SemaphoreType.DMA,
            pltpu.SemaphoreType.DMA,
            pltpu.SemaphoreType.DMA,
        ],
    )
    def seg_sum(src_hbm, dst_hbm, table_hbm, zeros_hbm, out_hbm,
                src_v, dst_v, rows_v, src_t, dst_t, rows_t, acc_sh,
                semi0, semi1, semi2, semg0, semg1, sems0, sems1, semz):
        cid = lax.axis_index("c")
        sid = lax.axis_index("s")
        wid = sid * NC + cid
        row0 = sid * ROWS_PER_SUB
        sem_i = (semi0, semi1, semi2)
        sem_g = (semg0, semg1)
        sem_s = (sems0, sems1)
        ebase = wid * EDGES_PER_W

        def idx_fetch(ci, s, sem):
            pltpu.async_copy(src_hbm.at[pl.ds(ebase + ci * CHUNK, CHUNK)],
                             src_v.at[s], sem)
            pltpu.async_copy(dst_hbm.at[pl.ds(ebase + ci * CHUNK, CHUNK)],
                             dst_v.at[s], sem)

        def idx_wait(s, sem):
            pltpu.make_async_copy(src_hbm.at[pl.ds(0, CHUNK)],
                                  src_v.at[s], sem).wait()
            pltpu.make_async_copy(dst_hbm.at[pl.ds(0, CHUNK)],
                                  dst_v.at[s], sem).wait()

        def gather(b, s, sem):
            pltpu.async_copy(table_hbm.at[src_v.at[s]], rows_v.at[b], sem)

        def gather_wait(b, s, sem):
            pltpu.make_async_copy(table_hbm.at[src_v.at[s]],
                                  rows_v.at[b], sem).wait()

        def scatter(b, s, sem):
            pltpu.async_copy(rows_v.at[b], acc_sh.at[dst_v.at[s]], sem,
                             add=True)

        def scatter_wait(b, s, sem):
            pltpu.make_async_copy(rows_v.at[b], acc_sh.at[dst_v.at[s]],
                                  sem).wait()

        zcp = pltpu.async_copy(zeros_hbm.at[pl.ds(row0, ROWS_PER_SUB)],
                               acc_sh.at[pl.ds(row0, ROWS_PER_SUB)], semz)

        idx_fetch(0, 0, sem_i[0])
        idx_wait(0, sem_i[0])
        gather(0, 0, sem_g[0])
        idx_fetch(1, 1, sem_i[1])
        zcp.wait()
        plsc.subcore_barrier()

        def body(i, carry):
            for k in range(6):
                ci = i * 6 + k
                b, nb = k % 2, (k + 1) % 2
                s, ns, ps = k % 3, (k + 1) % 3, (k + 2) % 3

                @pl.when(ci + 1 < NCHUNK)
                def _():
                    idx_wait(ns, sem_i[ns])

                @pl.when(ci >= 1)
                def _():
                    scatter_wait(nb, ps, sem_s[nb])

                @pl.when(ci + 1 < NCHUNK)
                def _():
                    gather(nb, ns, sem_g[nb])

                gather_wait(b, s, sem_g[b])
                scatter(b, s, sem_s[b])

                @pl.when(ci + 2 < NCHUNK)
                def _():
                    idx_fetch(ci + 2, ps, sem_i[ps])
            return carry

        lax.fori_loop(0, NCHUNK // 6, body, 0)
        scatter_wait(1, 2, sem_s[1])
        tbase = ebase + NCHUNK * CHUNK
        pltpu.sync_copy(src_hbm.at[pl.ds(tbase, TAIL)], src_t)
        pltpu.sync_copy(dst_hbm.at[pl.ds(tbase, TAIL)], dst_t)
        pltpu.async_copy(table_hbm.at[src_t], rows_t, sem_g[0]).wait()
        pltpu.sync_copy(rows_t, acc_sh.at[dst_t], add=True)
        plsc.subcore_barrier()

        pltpu.sync_copy(acc_sh.at[pl.ds(row0, ROWS_PER_SUB)],
                        out_hbm.at[cid, pl.ds(row0, ROWS_PER_SUB)])

    return seg_sum


BR = 1000
GRID = N_NODES // BR


def _mm1bn_body(acc_ref, w_ref, b_ref, gamma_ref, beta_ref, h1_ref,
                h_vmem, stats_vmem):
    p = pl.program_id(0)
    i = pl.program_id(1)

    @pl.when(p == 0)
    def _():
        a = acc_ref[0] + acc_ref[1]
        h = jnp.dot(a, w_ref[...],
                    preferred_element_type=jnp.float32) + b_ref[...]
        h_vmem[pl.ds(i * BR, BR), :] = h

        @pl.when(i == 0)
        def _():
            stats_vmem[...] = jnp.zeros_like(stats_vmem)

        stats_vmem[0:1, :] += jnp.sum(h, axis=0, keepdims=True)
        stats_vmem[1:2, :] += jnp.sum(h * h, axis=0, keepdims=True)

    @pl.when(p == 1)
    def _():
        mean = stats_vmem[0:1, :] / N_NODES
        var = stats_vmem[1:2, :] / N_NODES - mean * mean
        rstd = lax.rsqrt(var + 1e-5)
        h = h_vmem[pl.ds(i * BR, BR), :]
        h1_ref[...] = jnp.maximum(
            (h - mean) * (rstd * gamma_ref[...]) + beta_ref[...], 0.0)


_mm1bn = pl.pallas_call(
    _mm1bn_body,
    grid=(2, GRID),
    in_specs=[
        pl.BlockSpec((NC, BR, DIM), lambda p, i: (0, i * (1 - p), 0)),
        pl.BlockSpec((DIM, DIM), lambda p, i: (0, 0)),
        pl.BlockSpec((1, DIM), lambda p, i: (0, 0)),
        pl.BlockSpec((1, DIM), lambda p, i: (0, 0)),
        pl.BlockSpec((1, DIM), lambda p, i: (0, 0)),
    ],
    out_specs=pl.BlockSpec((BR, DIM), lambda p, i: (i, 0)),
    out_shape=jax.ShapeDtypeStruct((N_NODES, DIM), jnp.float32),
    scratch_shapes=[
        pltpu.VMEM((N_NODES, DIM), jnp.float32),
        pltpu.VMEM((2, DIM), jnp.float32),
    ],
)


def _final_body(acc_ref, h1_ref, w2_ref, b2_ref, wfc_ref, bfc_ref, o_ref):
    a = acc_ref[0] + acc_ref[1]
    h2 = jnp.dot(a, w2_ref[...], preferred_element_type=jnp.float32) + b2_ref[...]
    hjk = jnp.maximum(h1_ref[...], h2)
    o_ref[...] = jnp.dot(hjk, wfc_ref[...],
                         preferred_element_type=jnp.float32) + bfc_ref[...]


_final = pl.pallas_call(
    _final_body,
    grid=(GRID,),
    in_specs=[
        pl.BlockSpec((NC, BR, DIM), lambda i: (0, i, 0)),
        pl.BlockSpec((BR, DIM), lambda i: (i, 0)),
        pl.BlockSpec((DIM, DIM), lambda i: (0, 0)),
        pl.BlockSpec((1, DIM), lambda i: (0, 0)),
        pl.BlockSpec((DIM, DIM), lambda i: (0, 0)),
        pl.BlockSpec((1, DIM), lambda i: (0, 0)),
    ],
    out_specs=pl.BlockSpec((BR, DIM), lambda i: (i, 0)),
    out_shape=jax.ShapeDtypeStruct((N_NODES, DIM), jnp.float32),
)


def kernel(x, edge_index, W1, b1, gamma, beta, W2, b2, Wfc, bfc):
    src = edge_index[0]
    dst = edge_index[1]
    zeros = jnp.zeros((N_PAD, DIM), jnp.float32)

    seg_sum = _make_segment_sum_sc()
    acc1 = seg_sum(src, dst, x, zeros)
    h1 = _mm1bn(acc1, W1, b1.reshape(1, DIM),
                gamma.reshape(1, DIM), beta.reshape(1, DIM))
    acc2 = seg_sum(src, dst, h1, zeros)
    out = _final(acc2, h1, W2, b2.reshape(1, DIM), Wfc, bfc.reshape(1, DIM))
    return out


# speedup vs baseline: 1.0576x; 1.0289x over previous
import functools

import jax
import jax.numpy as jnp
from jax import lax
from jax.experimental import pallas as pl
from jax.experimental.pallas import tpu as pltpu
from jax.experimental.pallas import tpu_sc as plsc

N_NODES = 10000
N_PAD = 10240
DIM = 128
NUM_EDGES = 320000

NC, NS = 2, 16
NW = NC * NS
EDGES_PER_W = NUM_EDGES // NW
CHUNK = 128
NCHUNK = EDGES_PER_W // CHUNK
TAIL = EDGES_PER_W - NCHUNK * CHUNK
ROWS_PER_SUB = N_PAD // NS

@functools.cache
def _make_segment_sum_sc():
    mesh = plsc.VectorSubcoreMesh(core_axis_name="c", subcore_axis_name="s",
                                  num_cores=NC, num_subcores=NS)

    @functools.partial(
        pl.kernel,
        out_type=jax.ShapeDtypeStruct((NC, N_PAD, DIM), jnp.float32),
        mesh=mesh,
        scratch_types=[
            pltpu.VMEM((3, CHUNK), jnp.int32),
            pltpu.VMEM((3, CHUNK), jnp.int32),
            pltpu.VMEM((2, CHUNK, DIM), jnp.float32),
            pltpu.VMEM((TAIL,), jnp.int32),
            pltpu.VMEM((TAIL,), jnp.int32),
            pltpu.VMEM((TAIL, DIM), jnp.float32),
            pltpu.VMEM_SHARED((N_PAD, DIM), jnp.float32),
            pltpu.SemaphoreType.DMA,
            pltpu.SemaphoreType.DMA,
            pltpu.SemaphoreType.DMA,
            pltpu.SemaphoreType.DMA,
            pltpu.SemaphoreType.DMA,
            pltpu.SemaphoreType.DMA,
            pltpu.SemaphoreType.DMA,
            pltpu.SemaphoreType.DMA,
        ],
    )
    def seg_sum(src_hbm, dst_hbm, table_hbm, zeros_hbm, out_hbm,
                src_v, dst_v, rows_v, src_t, dst_t, rows_t, acc_sh,
                semi0, semi1, semi2, semg0, semg1, sems0, sems1, semz):
        cid = lax.axis_index("c")
        sid = lax.axis_index("s")
        wid = sid * NC + cid
        row0 = sid * ROWS_PER_SUB
        sem_i = (semi0, semi1, semi2)
        sem_g = (semg0, semg1)
        sem_s = (sems0, sems1)
        ebase = wid * EDGES_PER_W

        def idx_fetch(ci, s, sem):
            pltpu.async_copy(src_hbm.at[pl.ds(ebase + ci * CHUNK, CHUNK)],
                             src_v.at[s], sem)
            pltpu.async_copy(dst_hbm.at[pl.ds(ebase + ci * CHUNK, CHUNK)],
                             dst_v.at[s], sem)

        def idx_wait(s, sem):
            pltpu.make_async_copy(src_hbm.at[pl.ds(0, CHUNK)],
                                  src_v.at[s], sem).wait()
            pltpu.make_async_copy(dst_hbm.at[pl.ds(0, CHUNK)],
                                  dst_v.at[s], sem).wait()

        def gather(b, s, sem):
            pltpu.async_copy(table_hbm.at[src_v.at[s]], rows_v.at[b], sem)

        def gather_wait(b, s, sem):
            pltpu.make_async_copy(table_hbm.at[src_v.at[s]],
                                  rows_v.at[b], sem).wait()

        def scatter(b, s, sem):
            pltpu.async_copy(rows_v.at[b], acc_sh.at[dst_v.at[s]], sem,
                             add=True)

        def scatter_wait(b, s, sem):
            pltpu.make_async_copy(rows_v.at[b], acc_sh.at[dst_v.at[s]],
                                  sem).wait()

        zcp = pltpu.async_copy(zeros_hbm.at[pl.ds(row0, ROWS_PER_SUB)],
                               acc_sh.at[pl.ds(row0, ROWS_PER_SUB)], semz)

        idx_fetch(0, 0, sem_i[0])
        idx_wait(0, sem_i[0])
        gather(0, 0, sem_g[0])
        idx_fetch(1, 1, sem_i[1])
        zcp.wait()
        plsc.subcore_barrier()

        def body(i, carry):
            for k in range(6):
                ci = i * 6 + k
                b, nb = k % 2, (k + 1) % 2
                s, ns, ps = k % 3, (k + 1) % 3, (k + 2) % 3

                @pl.when(ci + 1 < NCHUNK)
                def _():
                    idx_wait(ns, sem_i[ns])

                @pl.when(ci >= 1)
                def _():
                    scatter_wait(nb, ps, sem_s[nb])

                @pl.when(ci + 1 < NCHUNK)
                def _():
                    gather(nb, ns, sem_g[nb])

                gather_wait(b, s, sem_g[b])
                scatter(b, s, sem_s[b])

                @pl.when(ci + 2 < NCHUNK)
                def _():
                    idx_fetch(ci + 2, ps, sem_i[ps])
            return carry

        lax.fori_loop(0, NCHUNK // 6, body, 0)
        scatter_wait(1, 2, sem_s[1])
        tbase = ebase + NCHUNK * CHUNK
        pltpu.sync_copy(src_hbm.at[pl.ds(tbase, TAIL)], src_t)
        pltpu.sync_copy(dst_hbm.at[pl.ds(tbase, TAIL)], dst_t)
        pltpu.async_copy(table_hbm.at[src_t], rows_t, sem_g[0]).wait()
        pltpu.sync_copy(rows_t, acc_sh.at[dst_t], add=True)
        plsc.subcore_barrier()

        pltpu.sync_copy(acc_sh.at[pl.ds(row0, ROWS_PER_SUB)],
                        out_hbm.at[cid, pl.ds(row0, ROWS_PER_SUB)])

    return seg_sum


BR = 2000
GRID = N_NODES // BR


def _mm1bn_body(acc_ref, w_ref, b_ref, gamma_ref, beta_ref, h1_ref,
                h_vmem, stats_vmem):
    p = pl.program_id(0)
    i = pl.program_id(1)

    @pl.when(p == 0)
    def _():
        a = acc_ref[0] + acc_ref[1]
        h = jnp.dot(a, w_ref[...],
                    preferred_element_type=jnp.float32) + b_ref[...]
        h_vmem[pl.ds(i * BR, BR), :] = h

        @pl.when(i == 0)
        def _():
            stats_vmem[...] = jnp.zeros_like(stats_vmem)

        stats_vmem[0:1, :] += jnp.sum(h, axis=0, keepdims=True)
        stats_vmem[1:2, :] += jnp.sum(h * h, axis=0, keepdims=True)

    @pl.when(p == 1)
    def _():
        mean = stats_vmem[0:1, :] / N_NODES
        var = stats_vmem[1:2, :] / N_NODES - mean * mean
        rstd = lax.rsqrt(var + 1e-5)
        h = h_vmem[pl.ds(i * BR, BR), :]
        h1_ref[...] = jnp.maximum(
            (h - mean) * (rstd * gamma_ref[...]) + beta_ref[...], 0.0)


_mm1bn = pl.pallas_call(
    _mm1bn_body,
    grid=(2, GRID),
    in_specs=[
        pl.BlockSpec((NC, BR, DIM), lambda p, i: (0, i * (1 - p), 0)),
        pl.BlockSpec((DIM, DIM), lambda p, i: (0, 0)),
        pl.BlockSpec((1, DIM), lambda p, i: (0, 0)),
        pl.BlockSpec((1, DIM), lambda p, i: (0, 0)),
        pl.BlockSpec((1, DIM), lambda p, i: (0, 0)),
    ],
    out_specs=pl.BlockSpec((BR, DIM), lambda p, i: (i, 0)),
    out_shape=jax.ShapeDtypeStruct((N_NODES, DIM), jnp.float32),
    scratch_shapes=[
        pltpu.VMEM((N_NODES, DIM), jnp.float32),
        pltpu.VMEM((2, DIM), jnp.float32),
    ],
)


def _final_body(acc_ref, h1_ref, w2_ref, b2_ref, wfc_ref, bfc_ref, o_ref):
    a = acc_ref[0] + acc_ref[1]
    h2 = jnp.dot(a, w2_ref[...], preferred_element_type=jnp.float32) + b2_ref[...]
    hjk = jnp.maximum(h1_ref[...], h2)
    o_ref[...] = jnp.dot(hjk, wfc_ref[...],
                         preferred_element_type=jnp.float32) + bfc_ref[...]


_final = pl.pallas_call(
    _final_body,
    grid=(GRID,),
    in_specs=[
        pl.BlockSpec((NC, BR, DIM), lambda i: (0, i, 0)),
        pl.BlockSpec((BR, DIM), lambda i: (i, 0)),
        pl.BlockSpec((DIM, DIM), lambda i: (0, 0)),
        pl.BlockSpec((1, DIM), lambda i: (0, 0)),
        pl.BlockSpec((DIM, DIM), lambda i: (0, 0)),
        pl.BlockSpec((1, DIM), lambda i: (0, 0)),
    ],
    out_specs=pl.BlockSpec((BR, DIM), lambda i: (i, 0)),
    out_shape=jax.ShapeDtypeStruct((N_NODES, DIM), jnp.float32),
)


def kernel(x, edge_index, W1, b1, gamma, beta, W2, b2, Wfc, bfc):
    src = edge_index[0]
    dst = edge_index[1]
    zeros = jnp.zeros((N_PAD, DIM), jnp.float32)

    seg_sum = _make_segment_sum_sc()
    acc1 = seg_sum(src, dst, x, zeros)
    h1 = _mm1bn(acc1, W1, b1.reshape(1, DIM),
                gamma.reshape(1, DIM), beta.reshape(1, DIM))
    acc2 = seg_sum(src, dst, h1, zeros)
    out = _final(acc2, h1, W2, b2.reshape(1, DIM), Wfc, bfc.reshape(1, DIM))
    return out


# speedup vs baseline: 1.0651x; 1.0071x over previous
import functools

import jax
import jax.numpy as jnp
from jax import lax
from jax.experimental import pallas as pl
from jax.experimental.pallas import tpu as pltpu
from jax.experimental.pallas import tpu_sc as plsc

N_NODES = 10000
N_PAD = 10240
DIM = 128
NUM_EDGES = 320000

NC, NS = 2, 16
NW = NC * NS
EDGES_PER_W = NUM_EDGES // NW
CHUNK = 128
NCHUNK = EDGES_PER_W // CHUNK
TAIL = EDGES_PER_W - NCHUNK * CHUNK
ROWS_PER_SUB = N_PAD // NS

@functools.cache
def _make_segment_sum_sc():
    mesh = plsc.VectorSubcoreMesh(core_axis_name="c", subcore_axis_name="s",
                                  num_cores=NC, num_subcores=NS)

    @functools.partial(
        pl.kernel,
        out_type=jax.ShapeDtypeStruct((NC, N_PAD, DIM), jnp.float32),
        mesh=mesh,
        scratch_types=[
            pltpu.VMEM((3, CHUNK), jnp.int32),
            pltpu.VMEM((3, CHUNK), jnp.int32),
            pltpu.VMEM((2, CHUNK, DIM), jnp.float32),
            pltpu.VMEM((TAIL,), jnp.int32),
            pltpu.VMEM((TAIL,), jnp.int32),
            pltpu.VMEM((TAIL, DIM), jnp.float32),
            pltpu.VMEM_SHARED((N_PAD, DIM), jnp.float32),
            pltpu.SemaphoreType.DMA,
            pltpu.SemaphoreType.DMA,
            pltpu.SemaphoreType.DMA,
            pltpu.SemaphoreType.DMA,
            pltpu.SemaphoreType.DMA,
            pltpu.SemaphoreType.DMA,
            pltpu.SemaphoreType.DMA,
            pltpu.SemaphoreType.DMA,
        ],
    )
    def seg_sum(src_hbm, dst_hbm, table_hbm, zeros_hbm, out_hbm,
                src_v, dst_v, rows_v, src_t, dst_t, rows_t, acc_sh,
                semi0, semi1, semi2, semg0, semg1, sems0, sems1, semz):
        cid = lax.axis_index("c")
        sid = lax.axis_index("s")
        wid = sid * NC + cid
        row0 = sid * ROWS_PER_SUB
        sem_i = (semi0, semi1, semi2)
        sem_g = (semg0, semg1)
        sem_s = (sems0, sems1)
        ebase = wid * EDGES_PER_W

        def idx_fetch(ci, s, sem):
            pltpu.async_copy(src_hbm.at[pl.ds(ebase + ci * CHUNK, CHUNK)],
                             src_v.at[s], sem)
            pltpu.async_copy(dst_hbm.at[pl.ds(ebase + ci * CHUNK, CHUNK)],
                             dst_v.at[s], sem)

        def idx_wait(s, sem):
            pltpu.make_async_copy(src_hbm.at[pl.ds(0, CHUNK)],
                                  src_v.at[s], sem).wait()
            pltpu.make_async_copy(dst_hbm.at[pl.ds(0, CHUNK)],
                                  dst_v.at[s], sem).wait()

        def gather(b, s, sem):
            pltpu.async_copy(table_hbm.at[src_v.at[s]], rows_v.at[b], sem)

        def gather_wait(b, s, sem):
            pltpu.make_async_copy(table_hbm.at[src_v.at[s]],
                                  rows_v.at[b], sem).wait()

        def scatter(b, s, sem):
            pltpu.async_copy(rows_v.at[b], acc_sh.at[dst_v.at[s]], sem,
                             add=True)

        def scatter_wait(b, s, sem):
            pltpu.make_async_copy(rows_v.at[b], acc_sh.at[dst_v.at[s]],
                                  sem).wait()

        zcp = pltpu.async_copy(zeros_hbm.at[pl.ds(row0, ROWS_PER_SUB)],
                               acc_sh.at[pl.ds(row0, ROWS_PER_SUB)], semz)

        idx_fetch(0, 0, sem_i[0])
        idx_wait(0, sem_i[0])
        gather(0, 0, sem_g[0])
        idx_fetch(1, 1, sem_i[1])
        zcp.wait()
        plsc.subcore_barrier()

        def body(i, carry):
            for k in range(6):
                ci = i * 6 + k
                b, nb = k % 2, (k + 1) % 2
                s, ns, ps = k % 3, (k + 1) % 3, (k + 2) % 3

                @pl.when(ci + 1 < NCHUNK)
                def _():
                    idx_wait(ns, sem_i[ns])

                @pl.when(ci >= 1)
                def _():
                    scatter_wait(nb, ps, sem_s[nb])

                @pl.when(ci + 1 < NCHUNK)
                def _():
                    gather(nb, ns, sem_g[nb])

                gather_wait(b, s, sem_g[b])
                scatter(b, s, sem_s[b])

                @pl.when(ci + 2 < NCHUNK)
                def _():
                    idx_fetch(ci + 2, ps, sem_i[ps])
            return carry

        lax.fori_loop(0, NCHUNK // 6, body, 0)
        scatter_wait(1, 2, sem_s[1])
        tbase = ebase + NCHUNK * CHUNK
        pltpu.sync_copy(src_hbm.at[pl.ds(tbase, TAIL)], src_t)
        pltpu.sync_copy(dst_hbm.at[pl.ds(tbase, TAIL)], dst_t)
        pltpu.async_copy(table_hbm.at[src_t], rows_t, sem_g[0]).wait()
        pltpu.sync_copy(rows_t, acc_sh.at[dst_t], add=True)
        plsc.subcore_barrier()

        pltpu.sync_copy(acc_sh.at[pl.ds(row0, ROWS_PER_SUB)],
                        out_hbm.at[cid, pl.ds(row0, ROWS_PER_SUB)])

    return seg_sum


BR = 5000
GRID = N_NODES // BR


def _mm1bn_body(acc_ref, w_ref, b_ref, gamma_ref, beta_ref, h1_ref,
                h_vmem, stats_vmem):
    p = pl.program_id(0)
    i = pl.program_id(1)

    @pl.when(p == 0)
    def _():
        a = acc_ref[0] + acc_ref[1]
        h = jnp.dot(a, w_ref[...],
                    preferred_element_type=jnp.float32) + b_ref[...]
        h_vmem[pl.ds(i * BR, BR), :] = h

        @pl.when(i == 0)
        def _():
            stats_vmem[...] = jnp.zeros_like(stats_vmem)

        stats_vmem[0:1, :] += jnp.sum(h, axis=0, keepdims=True)
        stats_vmem[1:2, :] += jnp.sum(h * h, axis=0, keepdims=True)

    @pl.when(p == 1)
    def _():
        mean = stats_vmem[0:1, :] / N_NODES
        var = stats_vmem[1:2, :] / N_NODES - mean * mean
        rstd = lax.rsqrt(var + 1e-5)
        h = h_vmem[pl.ds(i * BR, BR), :]
        h1_ref[...] = jnp.maximum(
            (h - mean) * (rstd * gamma_ref[...]) + beta_ref[...], 0.0)


_mm1bn = pl.pallas_call(
    _mm1bn_body,
    grid=(2, GRID),
    in_specs=[
        pl.BlockSpec((NC, BR, DIM), lambda p, i: (0, i * (1 - p), 0)),
        pl.BlockSpec((DIM, DIM), lambda p, i: (0, 0)),
        pl.BlockSpec((1, DIM), lambda p, i: (0, 0)),
        pl.BlockSpec((1, DIM), lambda p, i: (0, 0)),
        pl.BlockSpec((1, DIM), lambda p, i: (0, 0)),
    ],
    out_specs=pl.BlockSpec((BR, DIM), lambda p, i: (i, 0)),
    out_shape=jax.ShapeDtypeStruct((N_NODES, DIM), jnp.float32),
    scratch_shapes=[
        pltpu.VMEM((N_NODES, DIM), jnp.float32),
        pltpu.VMEM((2, DIM), jnp.float32),
    ],
)


def _final_body(acc_ref, h1_ref, w2_ref, b2_ref, wfc_ref, bfc_ref, o_ref):
    a = acc_ref[0] + acc_ref[1]
    h2 = jnp.dot(a, w2_ref[...], preferred_element_type=jnp.float32) + b2_ref[...]
    hjk = jnp.maximum(h1_ref[...], h2)
    o_ref[...] = jnp.dot(hjk, wfc_ref[...],
                         preferred_element_type=jnp.float32) + bfc_ref[...]


_final = pl.pallas_call(
    _final_body,
    grid=(GRID,),
    in_specs=[
        pl.BlockSpec((NC, BR, DIM), lambda i: (0, i, 0)),
        pl.BlockSpec((BR, DIM), lambda i: (i, 0)),
        pl.BlockSpec((DIM, DIM), lambda i: (0, 0)),
        pl.BlockSpec((1, DIM), lambda i: (0, 0)),
        pl.BlockSpec((DIM, DIM), lambda i: (0, 0)),
        pl.BlockSpec((1, DIM), lambda i: (0, 0)),
    ],
    out_specs=pl.BlockSpec((BR, DIM), lambda i: (i, 0)),
    out_shape=jax.ShapeDtypeStruct((N_NODES, DIM), jnp.float32),
)


def kernel(x, edge_index, W1, b1, gamma, beta, W2, b2, Wfc, bfc):
    src = edge_index[0]
    dst = edge_index[1]
    zeros = jnp.zeros((N_PAD, DIM), jnp.float32)

    seg_sum = _make_segment_sum_sc()
    acc1 = seg_sum(src, dst, x, zeros)
    h1 = _mm1bn(acc1, W1, b1.reshape(1, DIM),
                gamma.reshape(1, DIM), beta.reshape(1, DIM))
    acc2 = seg_sum(src, dst, h1, zeros)
    out = _final(acc2, h1, W2, b2.reshape(1, DIM), Wfc, bfc.reshape(1, DIM))
    return out
